# Initial kernel scaffold; baseline (speedup 1.0000x reference)
import jax
import jax.numpy as jnp
from jax.experimental import pallas as pl


def kernel(x):
    raise NotImplementedError("write your pallas kernel here")


# speedup vs baseline: 3.8865x; 3.8865x over previous
import jax
import jax.numpy as jnp
from jax.experimental import pallas as pl

_MAX_SPIKE = 100.0
_P = 0.1


def _spike_consts(B, T, dtype):
    key = jax.random.key(42)
    k1, k2, k3 = jax.random.split(key, 3)
    probas = jax.random.uniform(k1, (B,), dtype=jnp.float32)
    mask = probas > (1.0 - _P)
    pos = jax.random.randint(k2, (B,), 0, T - 2)
    sign = jnp.where(jax.random.randint(k3, (), 0, 2) == 0, -1.0, 1.0).astype(dtype)
    vals = jnp.where(mask, sign * _MAX_SPIKE, 0.0).astype(dtype)
    return pos, vals


def _body(pos_ref, val_ref, x_ref, o_ref):
    j = pl.program_id(1)
    bc = x_ref.shape[-1]
    cols = jax.lax.broadcasted_iota(jnp.int32, x_ref.shape, 1) + j * bc
    o_ref[...] = x_ref[...] + jnp.where(cols == pos_ref[...], val_ref[...], 0.0)


def kernel(x):
    B, T = x.shape
    pos, vals = _spike_consts(B, T, x.dtype)
    BR, BC = 128, 8192
    grid = (B // BR, T // BC)
    return pl.pallas_call(
        _body,
        grid=grid,
        in_specs=[
            pl.BlockSpec((BR, 1), lambda i, j: (i, 0)),
            pl.BlockSpec((BR, 1), lambda i, j: (i, 0)),
            pl.BlockSpec((BR, BC), lambda i, j: (i, j)),
        ],
        out_specs=pl.BlockSpec((BR, BC), lambda i, j: (i, j)),
        out_shape=jax.ShapeDtypeStruct((B, T), x.dtype),
    )(pos[:, None], vals[:, None], x)
